# baseline (speedup 1.0000x reference)
import functools

import jax
import jax.numpy as jnp
from jax import lax
from jax.experimental import pallas as pl
from jax.experimental.pallas import tpu as pltpu
from jax.experimental.pallas import tpu_sc as plsc

_B, _A, _F = 1024, 128, 75
_C = 16
_H = 33
_OUT = 64
_NW = 32
_MPW = _B // _NW
_MPS = 4
_NSUB = _MPW // _MPS
_NPAIR = _NSUB // 2
_L = 16
_DNUMS = jax.lax.GatherDimensionNumbers(
    offset_dims=(), collapsed_slice_dims=(0,), start_index_map=(0,))


def _make_sc():
    mesh = plsc.VectorSubcoreMesh(core_axis_name="c", subcore_axis_name="s")

    @functools.partial(
        pl.kernel,
        mesh=mesh,
        out_type=jax.ShapeDtypeStruct((_B, _OUT, _A), jnp.float32),
        scratch_types=[
            pltpu.VMEM((_C, _MPS, _A), jnp.float32),
            pltpu.VMEM((31, _MPS, _A), jnp.float32),
            pltpu.VMEM((2, _MPS, _OUT, _A), jnp.float32),
            pltpu.VMEM((_H, _C), jnp.float32),
            pltpu.VMEM((_L,), jnp.int32),
            pltpu.SemaphoreType.DMA,
            pltpu.SemaphoreType.DMA,
        ],
    )
    def _sc(xt_hbm, wf_hbm, n_hbm, out_hbm, cls_v, oth_v, out_v, w_v, n_v,
            out_s0, out_s1):
        wid = lax.axis_index("s") * 2 + lax.axis_index("c")
        pltpu.sync_copy(wf_hbm, w_v)
        pltpu.sync_copy(n_hbm, n_v)
        nvec = n_v[...]
        wcols = [w_v[f, pl.ds(0, _L)] for f in range(_H)]
        out_sems = (out_s0, out_s1)

        def stage_and_compute(s, slot):
            mol0 = wid * _MPW + s * _MPS
            pltpu.sync_copy(xt_hbm.at[pl.ds(0, _C), pl.ds(mol0, _MPS), :],
                            cls_v)
            pltpu.sync_copy(xt_hbm.at[pl.ds(44, 31), pl.ds(mol0, _MPS), :],
                            oth_v)

            @plsc.parallel_loop(0, _MPS * _A // _L, unroll=2)
            def group(g):
                m_i = g >> 3
                col = (g & 7) * _L
                rows = [cls_v[f, m_i, pl.ds(col, _L)] for f in range(_C)]
                mx = rows[0]
                for f in range(1, _C):
                    mx = jnp.maximum(mx, rows[f])
                p = jnp.zeros((_L,), jnp.int32)
                for f in range(_C - 1, -1, -1):
                    p = jnp.where(rows[f] == mx, f, p)
                molv = jax.lax.broadcast(mol0 + m_i, (_L,))
                scale = jnp.where(molv < nvec, 1.0, 0.0)
                pidx = p.reshape(_L, 1)
                for f in range(_H):
                    gf = lax.gather(
                        wcols[f], pidx, _DNUMS, slice_sizes=(1,),
                        mode=lax.GatherScatterMode.PROMISE_IN_BOUNDS)
                    out_v[slot, m_i, f, pl.ds(col, _L)] = gf * scale
                v0 = oth_v[0, m_i, pl.ds(col, _L)]
                ss = v0 * v0
                vs = [v0]
                for j in range(1, 31):
                    vj = oth_v[j, m_i, pl.ds(col, _L)]
                    vs.append(vj)
                    ss = ss + vj * vj
                ssc = jnp.maximum(ss, 1e-24)
                yi = jnp.int32(0x5F3759DF) - (
                    lax.bitcast_convert_type(ssc, jnp.int32) >> 1)
                y = lax.bitcast_convert_type(yi, jnp.float32)
                half = ssc * 0.5
                for _ in range(3):
                    y = y * (1.5 - half * y * y)
                y = y * scale
                for j in range(31):
                    out_v[slot, m_i, _H + j, pl.ds(col, _L)] = vs[j] * y

        def out_copy(s, slot):
            mol0 = wid * _MPW + s * _MPS
            return pltpu.make_async_copy(
                out_v.at[slot], out_hbm.at[pl.ds(mol0, _MPS), :, :],
                out_sems[slot])

        stage_and_compute(0, 0)
        out_copy(0, 0).start()
        stage_and_compute(1, 1)
        out_copy(1, 1).start()

        def pair(t, _unused):
            s0 = 2 * t
            s1 = s0 + 1
            out_copy(s0 - 2, 0).wait()
            stage_and_compute(s0, 0)
            out_copy(s0, 0).start()
            out_copy(s1 - 2, 1).wait()
            stage_and_compute(s1, 1)
            out_copy(s1, 1).start()
            return _unused

        lax.fori_loop(1, _NPAIR, pair, 0)
        out_copy(_NSUB - 2, 0).wait()
        out_copy(_NSUB - 1, 1).wait()

    return _sc


def kernel(molecule_atoms, W, N):
    xt = jnp.transpose(molecule_atoms, (2, 0, 1))
    wf = W.T
    nvec = jnp.full((_L,), N, jnp.int32)
    out = _make_sc()(xt, wf, nvec)
    return jnp.transpose(out, (0, 2, 1))


# speedup vs baseline: 1.2504x; 1.2504x over previous
import functools

import jax
import jax.numpy as jnp
from jax import lax
from jax.experimental import pallas as pl
from jax.experimental.pallas import tpu as pltpu
from jax.experimental.pallas import tpu_sc as plsc

_B, _A, _F = 1024, 128, 75
_C = 16
_H = 33
_OUT = 64
_NW = 32
_MPW = _B // _NW
_MPS = 8
_NSUB = _MPW // _MPS
_L = 16
_DNUMS = jax.lax.GatherDimensionNumbers(
    offset_dims=(), collapsed_slice_dims=(0,), start_index_map=(0,))


def _make_sc():
    mesh = plsc.VectorSubcoreMesh(core_axis_name="c", subcore_axis_name="s")

    @functools.partial(
        pl.kernel,
        mesh=mesh,
        out_type=jax.ShapeDtypeStruct((_B, _OUT, _A), jnp.float32),
        scratch_types=[
            pltpu.VMEM((_C, _MPS, _A), jnp.float32),
            pltpu.VMEM((31, _MPS, _A), jnp.float32),
            pltpu.VMEM((_MPS, _OUT, _A), jnp.float32),
            pltpu.VMEM((_H, _C), jnp.float32),
            pltpu.VMEM((_L,), jnp.int32),
            pltpu.SemaphoreType.DMA,
        ],
    )
    def _sc(xt_hbm, wf_hbm, n_hbm, out_hbm, cls_v, oth_v, out_v, w_v, n_v,
            out_sem):
        wid = lax.axis_index("s") * 2 + lax.axis_index("c")
        pltpu.sync_copy(wf_hbm, w_v)
        pltpu.sync_copy(n_hbm, n_v)
        nvec = n_v[...]
        wcols = [w_v[f, pl.ds(0, _L)] for f in range(_H)]

        def out_copy(s):
            mol0 = wid * _MPW + s * _MPS
            return pltpu.make_async_copy(
                out_v, out_hbm.at[pl.ds(mol0, _MPS), :, :], out_sem)

        def sub(s, first):
            mol0 = wid * _MPW + s * _MPS
            pltpu.sync_copy(xt_hbm.at[pl.ds(0, _C), pl.ds(mol0, _MPS), :], cls_v)
            pltpu.sync_copy(xt_hbm.at[pl.ds(44, 31), pl.ds(mol0, _MPS), :], oth_v)
            if not first:
                out_copy(s - 1).wait()

            @plsc.parallel_loop(0, _MPS * _A // _L, unroll=2)
            def group(g):
                m_i = g >> 3
                col = (g & 7) * _L
                rows = [cls_v[f, m_i, pl.ds(col, _L)] for f in range(_C)]
                mx = rows[0]
                for f in range(1, _C):
                    mx = jnp.maximum(mx, rows[f])
                p = jnp.zeros((_L,), jnp.int32)
                for f in range(_C - 1, -1, -1):
                    p = jnp.where(rows[f] == mx, f, p)
                molv = jax.lax.broadcast(mol0 + m_i, (_L,))
                scale = jnp.where(molv < nvec, 1.0, 0.0)
                pidx = p.reshape(_L, 1)
                for f in range(_H):
                    gf = lax.gather(
                        wcols[f], pidx, _DNUMS, slice_sizes=(1,),
                        mode=lax.GatherScatterMode.PROMISE_IN_BOUNDS)
                    out_v[m_i, f, pl.ds(col, _L)] = gf * scale
                v0 = oth_v[0, m_i, pl.ds(col, _L)]
                ss = v0 * v0
                vs = [v0]
                for j in range(1, 31):
                    vj = oth_v[j, m_i, pl.ds(col, _L)]
                    vs.append(vj)
                    ss = ss + vj * vj
                ssc = jnp.maximum(ss, 1e-24)
                yi = jnp.int32(0x5F3759DF) - (lax.bitcast_convert_type(ssc, jnp.int32) >> 1)
                y = lax.bitcast_convert_type(yi, jnp.float32)
                half = ssc * 0.5
                for _ in range(3):
                    y = y * (1.5 - half * y * y)
                y = y * scale
                for j in range(31):
                    out_v[m_i, _H + j, pl.ds(col, _L)] = vs[j] * y

            out_copy(s).start()
            return 0

        sub(0, True)
        lax.fori_loop(1, _NSUB, lambda s, c: sub(s, False), 0)
        out_copy(_NSUB - 1).wait()

    return _sc


def kernel(molecule_atoms, W, N):
    xt = jnp.transpose(molecule_atoms, (2, 0, 1))
    wf = W.T
    nvec = jnp.full((_L,), N, jnp.int32)
    out = _make_sc()(xt, wf, nvec)
    return jnp.transpose(out, (0, 2, 1))
